# Initial kernel scaffold; baseline (speedup 1.0000x reference)
import jax
import jax.numpy as jnp
from jax.experimental import pallas as pl


def kernel(hidden, logits, db_keys, db_values):
    raise NotImplementedError("write your pallas kernel here")


# speedup vs baseline: 1.6957x; 1.6957x over previous
import functools

import jax
import jax.numpy as jnp
from jax import lax
from jax.experimental import pallas as pl
from jax.experimental.pallas import tpu as pltpu
from jax.experimental.pallas import tpu_sc as plsc

_TOP_K = 32
_MIX = 0.25
_NEG = -1e30


def _topk_body(QB, KT, K, h_ref, k_ref, ti_ref, w_ref, s_ref):
    ki = pl.program_id(1)
    nk = pl.num_programs(1)
    keys = k_ref[...]
    ksq = jnp.sum(keys * keys, axis=1)
    qk = lax.dot_general(h_ref[...], keys, (((1,), (1,)), ((), ())),
                         preferred_element_type=jnp.float32)
    s_ref[:, pl.ds(ki * KT, KT)] = 2.0 * qk - ksq[None, :]

    @pl.when(ki == nk - 1)
    def _():
        lane32 = lax.broadcasted_iota(jnp.int32, (QB, _TOP_K), 1)

        def tbody(t, carry):
            tv, tix = carry
            iota = lax.broadcasted_iota(jnp.int32, (QB, K), 1)
            s = s_ref[...]
            m = jnp.max(s, axis=1, keepdims=True)
            idx = jnp.min(jnp.where(s == m, iota, K), axis=1, keepdims=True)
            s_ref[...] = jnp.where(iota == idx, _NEG, s)
            return (jnp.where(lane32 == t, m, tv),
                    jnp.where(lane32 == t, idx, tix))

        tv, tix = lax.fori_loop(
            0, _TOP_K, tbody,
            (jnp.zeros((QB, _TOP_K), jnp.float32),
             jnp.zeros((QB, _TOP_K), jnp.int32)))
        w = jnp.exp(tv - tv[:, 0:1])
        w = (_MIX / jnp.sum(w, axis=1, keepdims=True)) * w
        ti_ref[...] = tix
        w_ref[...] = w


def _run_topk(h, db_keys):
    Q, D = h.shape
    K = db_keys.shape[0]
    QB, KT = 64, 2048
    body = functools.partial(_topk_body, QB, KT, K)
    return pl.pallas_call(
        body,
        grid=(Q // QB, K // KT),
        in_specs=[
            pl.BlockSpec((QB, D), lambda qi, ki: (qi, 0)),
            pl.BlockSpec((KT, D), lambda qi, ki: (ki, 0)),
        ],
        out_specs=[
            pl.BlockSpec((QB, _TOP_K), lambda qi, ki: (qi, 0)),
            pl.BlockSpec((QB, _TOP_K), lambda qi, ki: (qi, 0)),
        ],
        out_shape=[
            jax.ShapeDtypeStruct((Q, _TOP_K), jnp.int32),
            jax.ShapeDtypeStruct((Q, _TOP_K), jnp.float32),
        ],
        scratch_shapes=[pltpu.VMEM((QB, K), jnp.float32)],
        compiler_params=pltpu.CompilerParams(
            dimension_semantics=("parallel", "arbitrary")),
    )(h, db_keys)


def _run_scatter(ti_flat, w_flat, db_values, Q, V):
    KDB = db_values.shape[0]
    NW = 32
    QW = Q // NW
    mesh = plsc.VectorSubcoreMesh(core_axis_name="c", subcore_axis_name="s")

    @functools.partial(
        pl.kernel, mesh=mesh,
        out_type=jax.ShapeDtypeStruct((Q, V), jnp.float32),
        compiler_params=pltpu.CompilerParams(
            use_tc_tiling_on_sc=False, needs_layout_passes=False),
        scratch_types=[
            pltpu.VMEM((KDB,), jnp.int32),
            pltpu.VMEM((QW * _TOP_K,), jnp.int32),
            pltpu.VMEM((QW * _TOP_K,), jnp.float32),
            pltpu.VMEM((V,), jnp.float32),
        ],
    )
    def sc_kernel(ti_hbm, w_hbm, dbv_hbm, out_hbm, dbv_v, idx_v, wv_v, row_v):
        wid = lax.axis_index("s") * 2 + lax.axis_index("c")
        base = wid * QW * _TOP_K
        pltpu.sync_copy(dbv_hbm, dbv_v)
        pltpu.sync_copy(ti_hbm.at[pl.ds(base, QW * _TOP_K)], idx_v)
        pltpu.sync_copy(w_hbm.at[pl.ds(base, QW * _TOP_K)], wv_v)

        zeros16 = jnp.zeros((16,), jnp.float32)

        def zbody(i, c):
            row_v[pl.ds(i * 16, 16)] = zeros16
            return c
        lax.fori_loop(0, V // 16, zbody, 0)

        lanes = lax.iota(jnp.int32, 16)

        def qbody(q, c):
            for half in range(_TOP_K // 16):
                off = q * _TOP_K + half * 16
                tok = plsc.load_gather(dbv_v, [idx_v[pl.ds(off, 16)]])
                wv = wv_v[pl.ds(off, 16)]
                for l in range(16):
                    plsc.addupdate_scatter(row_v, [tok], wv, mask=lanes == l)
            pltpu.sync_copy(row_v, out_hbm.at[wid * QW + q])
            for half in range(_TOP_K // 16):
                off = q * _TOP_K + half * 16
                tok = plsc.load_gather(dbv_v, [idx_v[pl.ds(off, 16)]])
                plsc.store_scatter(row_v, [tok], zeros16)
            return c
        lax.fori_loop(0, QW, qbody, 0)

    return sc_kernel(ti_flat, w_flat, db_values)


def _mix_body(lg_ref, ebd_ref, o_ref):
    lg = lg_ref[...]
    m = jnp.max(lg, axis=1, keepdims=True)
    e = jnp.exp(lg - m)
    z = jnp.sum(e, axis=1, keepdims=True)
    o_ref[...] = jnp.log(((1.0 - _MIX) / z) * e + ebd_ref[...])


def _run_mix(lg, ebd):
    Q, V = lg.shape
    QB = 32
    return pl.pallas_call(
        _mix_body,
        grid=(Q // QB,),
        in_specs=[
            pl.BlockSpec((QB, V), lambda qi: (qi, 0)),
            pl.BlockSpec((QB, V), lambda qi: (qi, 0)),
        ],
        out_specs=pl.BlockSpec((QB, V), lambda qi: (qi, 0)),
        out_shape=jax.ShapeDtypeStruct((Q, V), jnp.float32),
        compiler_params=pltpu.CompilerParams(
            dimension_semantics=("parallel",)),
    )(lg, ebd)


def kernel(hidden, logits, db_keys, db_values):
    B, T, D = hidden.shape
    V = logits.shape[-1]
    Q = B * T
    h = hidden.reshape(Q, D)
    lg = logits.reshape(Q, V)
    ti, w4 = _run_topk(h, db_keys)
    ebd4 = _run_scatter(ti.reshape(-1), w4.reshape(-1), db_values, Q, V)
    out = _run_mix(lg, ebd4)
    return out.reshape(B, T, V)


# speedup vs baseline: 3.5906x; 2.1175x over previous
import functools

import jax
import jax.numpy as jnp
from jax import lax
from jax.experimental import pallas as pl
from jax.experimental.pallas import tpu as pltpu
from jax.experimental.pallas import tpu_sc as plsc

_TOP_K = 32
_MIX = 0.25
_NEG = -1e30
_CH = 64


def _score_body(QB, KT, K, h_ref, k_ref, s_out, rows_ref, csel_ref, tval_ref,
                cm_ref):
    ki = pl.program_id(1)
    nk = pl.num_programs(1)
    qi = pl.program_id(0)
    CPT = KT // _CH
    keys = k_ref[...]
    ksq = jnp.sum(keys * keys, axis=1)
    qk = lax.dot_general(h_ref[...], keys, (((1,), (1,)), ((), ())),
                         preferred_element_type=jnp.float32)
    s = 2.0 * qk - ksq[None, :]
    s_out[...] = s
    cm_ref[ki] = jnp.max(s.reshape(QB, CPT, _CH), axis=2)

    @pl.when(ki == nk - 1)
    def _():
        C = K // _CH
        lane32 = lax.broadcasted_iota(jnp.int32, (QB, _TOP_K), 1)
        gid = (lax.broadcasted_iota(jnp.int32, (nk, QB, CPT), 0) * CPT
               + lax.broadcasted_iota(jnp.int32, (nk, QB, CPT), 2))

        def tbody(t, carry):
            tv, tix = carry
            cm = cm_ref[...]
            m = jnp.max(jnp.max(cm, axis=0), axis=1)
            sel = cm == m[None, :, None]
            idx = jnp.min(jnp.min(jnp.where(sel, gid, C), axis=0), axis=1)
            cm_ref[...] = jnp.where(gid == idx[None, :, None], _NEG, cm)
            return (jnp.where(lane32 == t, m[:, None], tv),
                    jnp.where(lane32 == t, idx[:, None], tix))

        tv, tix = lax.fori_loop(
            0, _TOP_K, tbody,
            (jnp.zeros((QB, _TOP_K), jnp.float32),
             jnp.zeros((QB, _TOP_K), jnp.int32)))
        csel_ref[...] = tix
        tval_ref[...] = tv
        qbase = qi * QB
        qrow = qbase + lax.broadcasted_iota(jnp.int32, (QB, _TOP_K), 0)
        rows_ref[...] = qrow * C + tix


def _run_scores(h, db_keys):
    Q, D = h.shape
    K = db_keys.shape[0]
    QB, KT = 64, 2048
    body = functools.partial(_score_body, QB, KT, K)
    return pl.pallas_call(
        body,
        grid=(Q // QB, K // KT),
        in_specs=[
            pl.BlockSpec((QB, D), lambda qi, ki: (qi, 0)),
            pl.BlockSpec((KT, D), lambda qi, ki: (ki, 0)),
        ],
        out_specs=[
            pl.BlockSpec((QB, KT), lambda qi, ki: (qi, ki)),
            pl.BlockSpec((QB, _TOP_K), lambda qi, ki: (qi, 0)),
            pl.BlockSpec((QB, _TOP_K), lambda qi, ki: (qi, 0)),
            pl.BlockSpec((QB, _TOP_K), lambda qi, ki: (qi, 0)),
        ],
        out_shape=[
            jax.ShapeDtypeStruct((Q, K), jnp.float32),
            jax.ShapeDtypeStruct((Q, _TOP_K), jnp.int32),
            jax.ShapeDtypeStruct((Q, _TOP_K), jnp.int32),
            jax.ShapeDtypeStruct((Q, _TOP_K), jnp.float32),
        ],
        scratch_shapes=[pltpu.VMEM((K // KT, QB, KT // _CH), jnp.float32)],
        compiler_params=pltpu.CompilerParams(
            dimension_semantics=("parallel", "arbitrary")),
    )(h, db_keys)


def _run_retrieve(s_tbl, rows_flat, cs_flat, t_flat, db_values, Q, V, C):
    KDB = db_values.shape[0]
    NW = 32
    QW = Q // NW
    NCAND = _TOP_K * _CH
    mesh = plsc.VectorSubcoreMesh(core_axis_name="c", subcore_axis_name="s")

    @functools.partial(
        pl.kernel, mesh=mesh,
        out_type=jax.ShapeDtypeStruct((Q, V), jnp.float32),
        compiler_params=pltpu.CompilerParams(
            use_tc_tiling_on_sc=False, needs_layout_passes=False),
        scratch_types=[
            pltpu.VMEM((KDB,), jnp.int32),
            pltpu.VMEM((QW, _TOP_K), jnp.int32),
            pltpu.VMEM((QW * _TOP_K * 16,), jnp.int32),
            pltpu.VMEM((QW * 16,), jnp.float32),
            pltpu.VMEM((_TOP_K, _CH), jnp.float32),
            pltpu.VMEM((NCAND + 16,), jnp.float32),
            pltpu.VMEM((NCAND + 16,), jnp.int32),
            pltpu.VMEM((V,), jnp.float32),
            pltpu.SemaphoreType.DMA,
        ],
    )
    def sc_kernel(s_hbm, rows_hbm, cs_hbm, t_hbm, dbv_hbm, out_hbm,
                  dbv_v, rid_v, cs_v, t_v, cand_v, vbuf, ibuf, row_v, sem):
        wid = lax.axis_index("s") * 2 + lax.axis_index("c")
        base = wid * QW * _TOP_K
        pltpu.sync_copy(dbv_hbm, dbv_v)
        pltpu.sync_copy(rows_hbm.at[pl.ds(wid * QW, QW)], rid_v)
        pltpu.sync_copy(cs_hbm.at[pl.ds(base * 16, QW * _TOP_K * 16)], cs_v)
        pltpu.sync_copy(t_hbm.at[pl.ds(wid * QW * 16, QW * 16)], t_v)

        zeros16 = jnp.zeros((16,), jnp.float32)
        negs16 = jnp.full((16,), _NEG, jnp.float32)
        ones16 = jnp.ones((16,), jnp.bool_)
        lanes = lax.iota(jnp.int32, 16)

        def zbody(i, c):
            row_v[pl.ds(i * 16, 16)] = zeros16
            return c
        lax.fori_loop(0, V // 16, zbody, 0)

        def qbody(q, c):
            qg = wid * QW + q
            pltpu.async_copy(s_hbm.at[rid_v.at[q]], cand_v, sem).wait()
            tq = t_v[pl.ds(q * 16, 16)]
            cnt = 0
            for r in range(_TOP_K):
                cbase = cs_v[pl.ds((q * _TOP_K + r) * 16, 16)] * _CH
                for j in range(_CH // 16):
                    v = cand_v[r, pl.ds(j * 16, 16)]
                    gi = (cbase + j * 16) + lanes
                    mask = v >= tq
                    plsc.store_compressed(vbuf.at[pl.ds(cnt, 16)], v, mask=mask)
                    plsc.store_compressed(ibuf.at[pl.ds(cnt, 16)], gi, mask=mask)
                    cnt = cnt + jnp.sum(mask.astype(jnp.int32))
            plsc.store_compressed(vbuf.at[pl.ds(cnt, 16)], negs16, mask=ones16)
            nv = lax.shift_right_logical(cnt + 15, 4)
            msel0 = negs16
            msel1 = negs16
            isel0 = jnp.zeros((16,), jnp.int32)
            isel1 = jnp.zeros((16,), jnp.int32)
            for t in range(_TOP_K):
                def mbody(j, acc):
                    return jnp.maximum(acc, vbuf[pl.ds(j * 16, 16)])
                acc = lax.fori_loop(0, nv, mbody, negs16)
                mval = jnp.max(acc)

                def ibody(j, imin):
                    vj = vbuf[pl.ds(j * 16, 16)]
                    ij = ibuf[pl.ds(j * 16, 16)]
                    cand = jnp.where(vj == mval, ij, jnp.int32(2**30))
                    return jnp.minimum(imin, jnp.min(cand))
                imin = lax.fori_loop(0, nv, ibody, jnp.int32(2**30))

                def xbody(j, c2):
                    vj = vbuf[pl.ds(j * 16, 16)]
                    ij = ibuf[pl.ds(j * 16, 16)]
                    vbuf[pl.ds(j * 16, 16)] = jnp.where(
                        (vj == mval) & (ij == imin), _NEG, vj)
                    return c2
                lax.fori_loop(0, nv, xbody, 0)
                if t < 16:
                    msel0 = jnp.where(lanes == t, mval, msel0)
                    isel0 = jnp.where(lanes == t, imin, isel0)
                else:
                    msel1 = jnp.where(lanes == t - 16, mval, msel1)
                    isel1 = jnp.where(lanes == t - 16, imin, isel1)
            m0 = jnp.max(msel0)
            e0 = jnp.exp(msel0 - m0)
            e1 = jnp.exp(msel1 - m0)
            denom = jnp.zeros((16,), jnp.float32) + (jnp.sum(e0) + jnp.sum(e1))
            scale = _MIX / denom
            w0 = e0 * scale
            w1 = e1 * scale
            tok0 = plsc.load_gather(dbv_v, [isel0])
            tok1 = plsc.load_gather(dbv_v, [isel1])
            for l in range(16):
                plsc.addupdate_scatter(row_v, [tok0], w0, mask=lanes == l)
                plsc.addupdate_scatter(row_v, [tok1], w1, mask=lanes == l)
            rb0 = plsc.load_gather(row_v, [tok0])
            dep = jnp.min(rb0 * 0.0).astype(jnp.int32)
            pltpu.sync_copy(row_v, out_hbm.at[qg + dep])
            plsc.store_scatter(row_v, [tok0], zeros16)
            plsc.store_scatter(row_v, [tok1], zeros16)
            return c
        lax.fori_loop(0, QW, qbody, 0)

    return sc_kernel(s_tbl, rows_flat, cs_flat, t_flat, db_values)


def _mix_body(lg_ref, ebd_ref, o_ref):
    lg = lg_ref[...]
    m = jnp.max(lg, axis=1, keepdims=True)
    e = jnp.exp(lg - m)
    z = jnp.sum(e, axis=1, keepdims=True)
    o_ref[...] = jnp.log(((1.0 - _MIX) / z) * e + ebd_ref[...])


def _run_mix(lg, ebd):
    Q, V = lg.shape
    QB = 32
    return pl.pallas_call(
        _mix_body,
        grid=(Q // QB,),
        in_specs=[
            pl.BlockSpec((QB, V), lambda qi: (qi, 0)),
            pl.BlockSpec((QB, V), lambda qi: (qi, 0)),
        ],
        out_specs=pl.BlockSpec((QB, V), lambda qi: (qi, 0)),
        out_shape=jax.ShapeDtypeStruct((Q, V), jnp.float32),
        compiler_params=pltpu.CompilerParams(
            dimension_semantics=("parallel",)),
    )(lg, ebd)


def kernel(hidden, logits, db_keys, db_values):
    B, T, D = hidden.shape
    V = logits.shape[-1]
    Q = B * T
    K = db_keys.shape[0]
    C = K // _CH
    h = hidden.reshape(Q, D)
    lg = logits.reshape(Q, V)
    s, rows, csel, tvals = _run_scores(h, db_keys)
    cs16 = jnp.broadcast_to(
        csel.reshape(-1, 1), (Q * _TOP_K, 16)).reshape(-1)
    t16 = jnp.broadcast_to(
        tvals[:, _TOP_K - 1:_TOP_K], (Q, 16)).reshape(-1)
    ebd4 = _run_retrieve(
        s.reshape(Q * C, _CH), rows, cs16, t16, db_values, Q, V, C)
    out = _run_mix(lg, ebd4)
    return out.reshape(B, T, V)


# speedup vs baseline: 3.8132x; 1.0620x over previous
import functools

import jax
import jax.numpy as jnp
from jax import lax
from jax.experimental import pallas as pl
from jax.experimental.pallas import tpu as pltpu
from jax.experimental.pallas import tpu_sc as plsc

_TOP_K = 32
_MIX = 0.25
_NEG = -1e30
_CH = 128


def _score_body(QB, KT, K, h_ref, k_ref, s_out, rows_ref, csel_ref, tval_ref,
                cm_ref, ksq_ref):
    ki = pl.program_id(1)
    nk = pl.num_programs(1)
    qi = pl.program_id(0)
    CPT = KT // _CH

    @pl.when(qi == 0)
    def _():
        keys = k_ref[...]
        ksq_ref[ki] = jnp.sum(keys * keys, axis=1)

    qk = lax.dot_general(h_ref[...], k_ref[...], (((1,), (1,)), ((), ())),
                         preferred_element_type=jnp.float32)
    s = 2.0 * qk - ksq_ref[ki][None, :]
    s_out[...] = s.reshape(QB * CPT, _CH)
    cm_ref[ki] = jnp.max(s.reshape(QB, CPT, _CH), axis=2)

    @pl.when(ki == nk - 1)
    def _():
        C = K // _CH
        lane32 = lax.broadcasted_iota(jnp.int32, (QB, _TOP_K), 1)
        gid = (lax.broadcasted_iota(jnp.int32, (nk, QB, CPT), 0) * CPT
               + lax.broadcasted_iota(jnp.int32, (nk, QB, CPT), 2))

        def tbody(t, carry):
            tv, tix = carry
            cm = cm_ref[...]
            m = jnp.max(jnp.max(cm, axis=0), axis=1)
            sel = cm == m[None, :, None]
            idx = jnp.min(jnp.min(jnp.where(sel, gid, C), axis=0), axis=1)
            cm_ref[...] = jnp.where(gid == idx[None, :, None], _NEG, cm)
            return (jnp.where(lane32 == t, m[:, None], tv),
                    jnp.where(lane32 == t, idx[:, None], tix))

        tv, tix = lax.fori_loop(
            0, _TOP_K, tbody,
            (jnp.zeros((QB, _TOP_K), jnp.float32),
             jnp.zeros((QB, _TOP_K), jnp.int32)))
        csel_ref[...] = tix
        tval_ref[...] = tv
        lq = lax.broadcasted_iota(jnp.int32, (QB, _TOP_K), 0)
        rows_ref[...] = (qi * (nk * QB * CPT)
                         + lax.shift_right_logical(tix, 4) * (QB * CPT)
                         + lq * CPT + jnp.bitwise_and(tix, CPT - 1))


def _run_scores(h, db_keys):
    Q, D = h.shape
    K = db_keys.shape[0]
    QB, KT = 64, 2048
    nk = K // KT
    CPT = KT // _CH
    body = functools.partial(_score_body, QB, KT, K)
    return pl.pallas_call(
        body,
        grid=(Q // QB, nk),
        in_specs=[
            pl.BlockSpec((QB, D), lambda qi, ki: (qi, 0)),
            pl.BlockSpec((KT, D), lambda qi, ki: (ki, 0)),
        ],
        out_specs=[
            pl.BlockSpec((QB * CPT, _CH),
                         lambda qi, ki, nk=nk: (qi * nk + ki, 0)),
            pl.BlockSpec((QB, _TOP_K), lambda qi, ki: (qi, 0)),
            pl.BlockSpec((QB, _TOP_K), lambda qi, ki: (qi, 0)),
            pl.BlockSpec((QB, _TOP_K), lambda qi, ki: (qi, 0)),
        ],
        out_shape=[
            jax.ShapeDtypeStruct((Q * (K // _CH), _CH), jnp.float32),
            jax.ShapeDtypeStruct((Q, _TOP_K), jnp.int32),
            jax.ShapeDtypeStruct((Q, _TOP_K), jnp.int32),
            jax.ShapeDtypeStruct((Q, _TOP_K), jnp.float32),
        ],
        scratch_shapes=[pltpu.VMEM((nk, QB, CPT), jnp.float32),
                        pltpu.VMEM((nk, KT), jnp.float32)],
        compiler_params=pltpu.CompilerParams(
            dimension_semantics=("parallel", "arbitrary")),
    )(h, db_keys)


def _run_retrieve(s_tbl, rows_flat, cs_flat, t_flat, db_values, Q, V, C):
    KDB = db_values.shape[0]
    NW = 32
    QW = Q // NW
    NCAND = _TOP_K * _CH
    mesh = plsc.VectorSubcoreMesh(core_axis_name="c", subcore_axis_name="s")

    @functools.partial(
        pl.kernel, mesh=mesh,
        out_type=jax.ShapeDtypeStruct((Q, V), jnp.float32),
        compiler_params=pltpu.CompilerParams(
            use_tc_tiling_on_sc=False, needs_layout_passes=False),
        scratch_types=[
            pltpu.VMEM((KDB,), jnp.int32),
            pltpu.VMEM((QW, _TOP_K), jnp.int32),
            pltpu.VMEM((QW * _TOP_K * 16,), jnp.int32),
            pltpu.VMEM((QW * 16,), jnp.float32),
            pltpu.VMEM((_TOP_K, _CH), jnp.float32),
            pltpu.VMEM((NCAND + 16,), jnp.float32),
            pltpu.VMEM((NCAND + 16,), jnp.int32),
            pltpu.VMEM((V,), jnp.float32),
            pltpu.SemaphoreType.DMA,
        ],
    )
    def sc_kernel(s_hbm, rows_hbm, cs_hbm, t_hbm, dbv_hbm, out_hbm,
                  dbv_v, rid_v, cs_v, t_v, cand_v, vbuf, ibuf, row_v, sem):
        wid = lax.axis_index("s") * 2 + lax.axis_index("c")
        base = wid * QW * _TOP_K
        pltpu.sync_copy(dbv_hbm, dbv_v)
        pltpu.sync_copy(rows_hbm.at[pl.ds(wid * QW, QW)], rid_v)
        pltpu.sync_copy(cs_hbm.at[pl.ds(base * 16, QW * _TOP_K * 16)], cs_v)
        pltpu.sync_copy(t_hbm.at[pl.ds(wid * QW * 16, QW * 16)], t_v)

        zeros16 = jnp.zeros((16,), jnp.float32)
        negs16 = jnp.full((16,), _NEG, jnp.float32)
        ones16 = jnp.ones((16,), jnp.bool_)
        lanes = lax.iota(jnp.int32, 16)

        def zbody(i, c):
            row_v[pl.ds(i * 16, 16)] = zeros16
            return c
        lax.fori_loop(0, V // 16, zbody, 0)

        def qbody(q, c):
            qg = wid * QW + q
            pltpu.async_copy(s_hbm.at[rid_v.at[q]], cand_v, sem).wait()
            tq = t_v[pl.ds(q * 16, 16)]
            cnt = 0
            for r in range(_TOP_K):
                cbase = cs_v[pl.ds((q * _TOP_K + r) * 16, 16)] * _CH
                for j in range(_CH // 16):
                    v = cand_v[r, pl.ds(j * 16, 16)]
                    gi = (cbase + j * 16) + lanes
                    mask = v >= tq
                    plsc.store_compressed(vbuf.at[pl.ds(cnt, 16)], v, mask=mask)
                    plsc.store_compressed(ibuf.at[pl.ds(cnt, 16)], gi, mask=mask)
                    cnt = cnt + jnp.sum(mask.astype(jnp.int32))
            plsc.store_compressed(vbuf.at[pl.ds(cnt, 16)], negs16, mask=ones16)
            nv = lax.shift_right_logical(cnt + 15, 4)
            msel0 = negs16
            msel1 = negs16
            isel0 = jnp.zeros((16,), jnp.int32)
            isel1 = jnp.zeros((16,), jnp.int32)
            for t in range(_TOP_K):
                def mbody(j, acc):
                    return jnp.maximum(acc, vbuf[pl.ds(j * 16, 16)])
                acc = lax.fori_loop(0, nv, mbody, negs16)
                mval = jnp.max(acc)

                def ibody(j, imin):
                    vj = vbuf[pl.ds(j * 16, 16)]
                    ij = ibuf[pl.ds(j * 16, 16)]
                    cand = jnp.where(vj == mval, ij, jnp.int32(2**30))
                    return jnp.minimum(imin, jnp.min(cand))
                imin = lax.fori_loop(0, nv, ibody, jnp.int32(2**30))

                def xbody(j, c2):
                    vj = vbuf[pl.ds(j * 16, 16)]
                    ij = ibuf[pl.ds(j * 16, 16)]
                    vbuf[pl.ds(j * 16, 16)] = jnp.where(
                        (vj == mval) & (ij == imin), _NEG, vj)
                    return c2
                lax.fori_loop(0, nv, xbody, 0)
                if t < 16:
                    msel0 = jnp.where(lanes == t, mval, msel0)
                    isel0 = jnp.where(lanes == t, imin, isel0)
                else:
                    msel1 = jnp.where(lanes == t - 16, mval, msel1)
                    isel1 = jnp.where(lanes == t - 16, imin, isel1)
            m0 = jnp.max(msel0)
            e0 = jnp.exp(msel0 - m0)
            e1 = jnp.exp(msel1 - m0)
            denom = jnp.zeros((16,), jnp.float32) + (jnp.sum(e0) + jnp.sum(e1))
            scale = _MIX / denom
            w0 = e0 * scale
            w1 = e1 * scale
            tok0 = plsc.load_gather(dbv_v, [isel0])
            tok1 = plsc.load_gather(dbv_v, [isel1])
            for l in range(16):
                plsc.addupdate_scatter(row_v, [tok0], w0, mask=lanes == l)
                plsc.addupdate_scatter(row_v, [tok1], w1, mask=lanes == l)
            rb0 = plsc.load_gather(row_v, [tok0])
            dep = jnp.min(rb0 * 0.0).astype(jnp.int32)
            pltpu.sync_copy(row_v, out_hbm.at[qg + dep])
            plsc.store_scatter(row_v, [tok0], zeros16)
            plsc.store_scatter(row_v, [tok1], zeros16)
            return c
        lax.fori_loop(0, QW, qbody, 0)

    return sc_kernel(s_tbl, rows_flat, cs_flat, t_flat, db_values)


def _mix_body(lg_ref, ebd_ref, o_ref):
    lg = lg_ref[...]
    m = jnp.max(lg, axis=1, keepdims=True)
    e = jnp.exp(lg - m)
    z = jnp.sum(e, axis=1, keepdims=True)
    o_ref[...] = jnp.log(((1.0 - _MIX) / z) * e + ebd_ref[...])


def _run_mix(lg, ebd):
    Q, V = lg.shape
    QB = 32
    return pl.pallas_call(
        _mix_body,
        grid=(Q // QB,),
        in_specs=[
            pl.BlockSpec((QB, V), lambda qi: (qi, 0)),
            pl.BlockSpec((QB, V), lambda qi: (qi, 0)),
        ],
        out_specs=pl.BlockSpec((QB, V), lambda qi: (qi, 0)),
        out_shape=jax.ShapeDtypeStruct((Q, V), jnp.float32),
        compiler_params=pltpu.CompilerParams(
            dimension_semantics=("parallel",)),
    )(lg, ebd)


def kernel(hidden, logits, db_keys, db_values):
    B, T, D = hidden.shape
    V = logits.shape[-1]
    Q = B * T
    K = db_keys.shape[0]
    C = K // _CH
    h = hidden.reshape(Q, D)
    lg = logits.reshape(Q, V)
    s, rows, csel, tvals = _run_scores(h, db_keys)
    cs16 = jnp.broadcast_to(
        csel.reshape(-1, 1), (Q * _TOP_K, 16)).reshape(-1)
    t16 = jnp.broadcast_to(
        tvals[:, _TOP_K - 1:_TOP_K], (Q, 16)).reshape(-1)
    ebd4 = _run_retrieve(s, rows, cs16, t16, db_values, Q, V, C)
    out = _run_mix(lg, ebd4)
    return out.reshape(B, T, V)


# speedup vs baseline: 4.1833x; 1.0971x over previous
import functools

import jax
import jax.numpy as jnp
from jax import lax
from jax.experimental import pallas as pl
from jax.experimental.pallas import tpu as pltpu
from jax.experimental.pallas import tpu_sc as plsc

_TOP_K = 32
_MIX = 0.25
_NEG = -1e30
_CH = 128


def _score_body(QB, KT, K, h_ref, k_ref, s_out, rows_ref, csel_ref, tval_ref,
                cm_ref, ksq_ref):
    ki = pl.program_id(1)
    nk = pl.num_programs(1)
    qi = pl.program_id(0)
    CPT = KT // _CH

    @pl.when(qi == 0)
    def _():
        keys = k_ref[...]
        ksq_ref[ki] = jnp.sum(keys * keys, axis=1)

    qk = lax.dot_general(h_ref[...], k_ref[...], (((1,), (1,)), ((), ())),
                         preferred_element_type=jnp.float32)
    s = 2.0 * qk - ksq_ref[ki][None, :]
    s_out[...] = s.reshape(QB * CPT, _CH)
    cm_ref[ki] = jnp.max(s.reshape(QB, CPT, _CH), axis=2)

    @pl.when(ki == nk - 1)
    def _():
        C = K // _CH
        lane32 = lax.broadcasted_iota(jnp.int32, (QB, _TOP_K), 1)
        gid = (lax.broadcasted_iota(jnp.int32, (nk, QB, CPT), 0) * CPT
               + lax.broadcasted_iota(jnp.int32, (nk, QB, CPT), 2))

        def tbody(t, carry):
            tv, tix = carry
            cm = cm_ref[...]
            m = jnp.max(jnp.max(cm, axis=0), axis=1)
            sel = cm == m[None, :, None]
            idx = jnp.min(jnp.min(jnp.where(sel, gid, C), axis=0), axis=1)
            cm_ref[...] = jnp.where(gid == idx[None, :, None], _NEG, cm)
            return (jnp.where(lane32 == t, m[:, None], tv),
                    jnp.where(lane32 == t, idx[:, None], tix))

        tv, tix = lax.fori_loop(
            0, _TOP_K, tbody,
            (jnp.zeros((QB, _TOP_K), jnp.float32),
             jnp.zeros((QB, _TOP_K), jnp.int32)))
        csel_ref[...] = tix
        tval_ref[...] = tv
        lq = lax.broadcasted_iota(jnp.int32, (QB, _TOP_K), 0)
        rows_ref[...] = (qi * (nk * QB * CPT)
                         + lax.shift_right_logical(tix, 4) * (QB * CPT)
                         + lq * CPT + jnp.bitwise_and(tix, CPT - 1))


def _run_scores(h, db_keys):
    Q, D = h.shape
    K = db_keys.shape[0]
    QB, KT = 64, 2048
    nk = K // KT
    CPT = KT // _CH
    body = functools.partial(_score_body, QB, KT, K)
    return pl.pallas_call(
        body,
        grid=(Q // QB, nk),
        in_specs=[
            pl.BlockSpec((QB, D), lambda qi, ki: (qi, 0)),
            pl.BlockSpec((KT, D), lambda qi, ki: (ki, 0)),
        ],
        out_specs=[
            pl.BlockSpec((QB * CPT, _CH),
                         lambda qi, ki, nk=nk: (qi * nk + ki, 0)),
            pl.BlockSpec((QB, _TOP_K), lambda qi, ki: (qi, 0)),
            pl.BlockSpec((QB, _TOP_K), lambda qi, ki: (qi, 0)),
            pl.BlockSpec((QB, _TOP_K), lambda qi, ki: (qi, 0)),
        ],
        out_shape=[
            jax.ShapeDtypeStruct((Q * (K // _CH), _CH), jnp.float32),
            jax.ShapeDtypeStruct((Q, _TOP_K), jnp.int32),
            jax.ShapeDtypeStruct((Q, _TOP_K), jnp.int32),
            jax.ShapeDtypeStruct((Q, _TOP_K), jnp.float32),
        ],
        scratch_shapes=[pltpu.VMEM((nk, QB, CPT), jnp.float32),
                        pltpu.VMEM((nk, KT), jnp.float32)],
        compiler_params=pltpu.CompilerParams(
            dimension_semantics=("parallel", "arbitrary")),
    )(h, db_keys)


def _run_retrieve(s_tbl, rows_flat, cs_flat, t_flat, db_values, Q, V, C):
    KDB = db_values.shape[0]
    NW = 32
    QW = Q // NW
    NCAND = _TOP_K * _CH
    mesh = plsc.VectorSubcoreMesh(core_axis_name="c", subcore_axis_name="s")

    @functools.partial(
        pl.kernel, mesh=mesh,
        out_type=jax.ShapeDtypeStruct((Q, V), jnp.float32),
        compiler_params=pltpu.CompilerParams(
            use_tc_tiling_on_sc=False, needs_layout_passes=False),
        scratch_types=[
            pltpu.VMEM((KDB,), jnp.int32),
            pltpu.VMEM((QW, _TOP_K), jnp.int32),
            pltpu.VMEM((QW * _TOP_K * 16,), jnp.int32),
            pltpu.VMEM((QW * 16,), jnp.float32),
            pltpu.VMEM((_TOP_K, _CH), jnp.float32),
            pltpu.VMEM((NCAND + 16,), jnp.float32),
            pltpu.VMEM((NCAND + 16,), jnp.int32),
            pltpu.VMEM((V,), jnp.float32),
            pltpu.SemaphoreType.DMA,
        ],
    )
    def sc_kernel(s_hbm, rows_hbm, cs_hbm, t_hbm, dbv_hbm, out_hbm,
                  dbv_v, rid_v, cs_v, t_v, cand_v, vbuf, ibuf, row_v, sem):
        wid = lax.axis_index("s") * 2 + lax.axis_index("c")
        base = wid * QW * _TOP_K
        pltpu.sync_copy(dbv_hbm, dbv_v)
        pltpu.sync_copy(rows_hbm.at[pl.ds(wid * QW, QW)], rid_v)
        pltpu.sync_copy(cs_hbm.at[pl.ds(base * 16, QW * _TOP_K * 16)], cs_v)
        pltpu.sync_copy(t_hbm.at[pl.ds(wid * QW * 16, QW * 16)], t_v)

        zeros16 = jnp.zeros((16,), jnp.float32)
        negs16 = jnp.full((16,), _NEG, jnp.float32)
        ones16 = jnp.ones((16,), jnp.bool_)
        lanes = lax.iota(jnp.int32, 16)

        def zbody(i, c):
            row_v[pl.ds(i * 16, 16)] = zeros16
            return c
        lax.fori_loop(0, V // 16, zbody, 0)

        def qbody(q, c):
            qg = wid * QW + q
            pltpu.async_copy(s_hbm.at[rid_v.at[q]], cand_v, sem).wait()
            tq = t_v[pl.ds(q * 16, 16)]
            cnt = 0
            for r in range(_TOP_K):
                cbase = cs_v[pl.ds((q * _TOP_K + r) * 16, 16)] * _CH
                for j in range(_CH // 16):
                    v = cand_v[r, pl.ds(j * 16, 16)]
                    gi = (cbase + j * 16) + lanes
                    mask = v >= tq
                    plsc.store_compressed(vbuf.at[pl.ds(cnt, 16)], v, mask=mask)
                    plsc.store_compressed(ibuf.at[pl.ds(cnt, 16)], gi, mask=mask)
                    cnt = cnt + jnp.sum(mask.astype(jnp.int32))
            plsc.store_compressed(vbuf.at[pl.ds(cnt, 16)], negs16, mask=ones16)
            nv = lax.shift_right_logical(cnt + 15, 4)
            msel0 = negs16
            msel1 = negs16
            isel0 = jnp.zeros((16,), jnp.int32)
            isel1 = jnp.zeros((16,), jnp.int32)
            for t in range(_TOP_K):
                def mbody(j, acc):
                    return jnp.maximum(acc, vbuf[pl.ds(j * 16, 16)])
                acc = lax.fori_loop(0, nv, mbody, negs16)
                mval = jnp.max(acc)

                def ibody(j, imin):
                    vj = vbuf[pl.ds(j * 16, 16)]
                    ij = ibuf[pl.ds(j * 16, 16)]
                    cand = jnp.where(vj == mval, ij, jnp.int32(2**30))
                    return jnp.minimum(imin, jnp.min(cand))
                imin = lax.fori_loop(0, nv, ibody, jnp.int32(2**30))

                def xbody(j, c2):
                    vj = vbuf[pl.ds(j * 16, 16)]
                    ij = ibuf[pl.ds(j * 16, 16)]
                    vbuf[pl.ds(j * 16, 16)] = jnp.where(
                        (vj == mval) & (ij == imin), _NEG, vj)
                    return c2
                lax.fori_loop(0, nv, xbody, 0)
                if t < 16:
                    msel0 = jnp.where(lanes == t, mval, msel0)
                    isel0 = jnp.where(lanes == t, imin, isel0)
                else:
                    msel1 = jnp.where(lanes == t - 16, mval, msel1)
                    isel1 = jnp.where(lanes == t - 16, imin, isel1)
            m0 = jnp.max(msel0)
            e0 = jnp.exp(msel0 - m0)
            e1 = jnp.exp(msel1 - m0)
            denom = jnp.zeros((16,), jnp.float32) + (jnp.sum(e0) + jnp.sum(e1))
            scale = _MIX / denom
            w0 = e0 * scale
            w1 = e1 * scale
            tok0 = plsc.load_gather(dbv_v, [isel0])
            tok1 = plsc.load_gather(dbv_v, [isel1])
            for l in range(16):
                plsc.addupdate_scatter(row_v, [tok0], w0, mask=lanes == l)
                plsc.addupdate_scatter(row_v, [tok1], w1, mask=lanes == l)
            rb0 = plsc.load_gather(row_v, [tok0])
            dep = jnp.min(rb0 * 0.0).astype(jnp.int32)
            pltpu.sync_copy(row_v, out_hbm.at[qg + dep])
            plsc.store_scatter(row_v, [tok0], zeros16)
            plsc.store_scatter(row_v, [tok1], zeros16)
            return c
        lax.fori_loop(0, QW, qbody, 0)

    return sc_kernel(s_tbl, rows_flat, cs_flat, t_flat, db_values)


def _mix_body(lg_ref, ea_ref, eb_ref, o_ref):
    nh = pl.num_programs(0) // 2
    qi = pl.program_id(0)
    lg = lg_ref[...]
    m = jnp.max(lg, axis=1, keepdims=True)
    e = jnp.exp(lg - m)
    z = jnp.sum(e, axis=1, keepdims=True)
    mbd = ((1.0 - _MIX) / z) * e

    @pl.when(qi < nh)
    def _():
        o_ref[...] = jnp.log(mbd + ea_ref[...])

    @pl.when(qi >= nh)
    def _():
        o_ref[...] = jnp.log(mbd + eb_ref[...])


def _run_mix(lg, ebd_a, ebd_b):
    Q, V = lg.shape
    QB = 32
    nh = Q // QB // 2
    return pl.pallas_call(
        _mix_body,
        grid=(Q // QB,),
        in_specs=[
            pl.BlockSpec((QB, V), lambda qi: (qi, 0)),
            pl.BlockSpec((QB, V), lambda qi, nh=nh: (qi % nh, 0)),
            pl.BlockSpec((QB, V), lambda qi, nh=nh: (qi % nh, 0)),
        ],
        out_specs=pl.BlockSpec((QB, V), lambda qi: (qi, 0)),
        out_shape=jax.ShapeDtypeStruct((Q, V), jnp.float32),
        compiler_params=pltpu.CompilerParams(
            dimension_semantics=("arbitrary",)),
    )(lg, ebd_a, ebd_b)


def _half_pipeline(h_half, db_keys, db_values, V):
    Qh, _ = h_half.shape
    K = db_keys.shape[0]
    C = K // _CH
    s, rows, csel, tvals = _run_scores(h_half, db_keys)
    cs16 = jnp.broadcast_to(
        csel.reshape(-1, 1), (Qh * _TOP_K, 16)).reshape(-1)
    t16 = jnp.broadcast_to(
        tvals[:, _TOP_K - 1:_TOP_K], (Qh, 16)).reshape(-1)
    return _run_retrieve(s, rows, cs16, t16, db_values, Qh, V, C)


def kernel(hidden, logits, db_keys, db_values):
    B, T, D = hidden.shape
    V = logits.shape[-1]
    Q = B * T
    h = hidden.reshape(Q, D)
    lg = logits.reshape(Q, V)
    ebd_a = _half_pipeline(h[:Q // 2], db_keys, db_values, V)
    ebd_b = _half_pipeline(h[Q // 2:], db_keys, db_values, V)
    out = _run_mix(lg, ebd_a, ebd_b)
    return out.reshape(B, T, V)


# speedup vs baseline: 4.2026x; 1.0046x over previous
import functools

import jax
import jax.numpy as jnp
from jax import lax
from jax.experimental import pallas as pl
from jax.experimental.pallas import tpu as pltpu
from jax.experimental.pallas import tpu_sc as plsc

_TOP_K = 32
_MIX = 0.25
_NEG = -1e30
_CH = 128


def _score_body(QB, KT, K, h_ref, k_ref, s_out, rows_ref, csel_ref, tval_ref,
                cm_ref, ksq_ref):
    ki = pl.program_id(1)
    nk = pl.num_programs(1)
    qi = pl.program_id(0)
    CPT = KT // _CH

    @pl.when(qi == 0)
    def _():
        keys = k_ref[...]
        ksq_ref[ki] = jnp.sum(keys * keys, axis=1)

    qk = lax.dot_general(h_ref[...], k_ref[...], (((1,), (1,)), ((), ())),
                         preferred_element_type=jnp.float32)
    s = 2.0 * qk - ksq_ref[ki][None, :]
    s_out[...] = s.reshape(QB * CPT, _CH)
    cm_ref[ki] = jnp.max(s.reshape(QB, CPT, _CH), axis=2)

    @pl.when(ki == nk - 1)
    def _():
        C = K // _CH
        lane32 = lax.broadcasted_iota(jnp.int32, (QB, _TOP_K), 1)
        gid = (lax.broadcasted_iota(jnp.int32, (nk, QB, CPT), 0) * CPT
               + lax.broadcasted_iota(jnp.int32, (nk, QB, CPT), 2))

        def tbody(t, carry):
            tv, tix = carry
            cm = cm_ref[...]
            m = jnp.max(jnp.max(cm, axis=0), axis=1)
            sel = cm == m[None, :, None]
            idx = jnp.min(jnp.min(jnp.where(sel, gid, C), axis=0), axis=1)
            cm_ref[...] = jnp.where(gid == idx[None, :, None], _NEG, cm)
            return (jnp.where(lane32 == t, m[:, None], tv),
                    jnp.where(lane32 == t, idx[:, None], tix))

        tv, tix = lax.fori_loop(
            0, _TOP_K, tbody,
            (jnp.zeros((QB, _TOP_K), jnp.float32),
             jnp.zeros((QB, _TOP_K), jnp.int32)))
        csel_ref[...] = tix
        tval_ref[...] = tv
        lq = lax.broadcasted_iota(jnp.int32, (QB, _TOP_K), 0)
        rows_ref[...] = (qi * (nk * QB * CPT)
                         + lax.shift_right_logical(tix, 4) * (QB * CPT)
                         + lq * CPT + jnp.bitwise_and(tix, CPT - 1))


def _run_scores(h, db_keys):
    Q, D = h.shape
    K = db_keys.shape[0]
    QB, KT = 64, 2048
    nk = K // KT
    CPT = KT // _CH
    body = functools.partial(_score_body, QB, KT, K)
    return pl.pallas_call(
        body,
        grid=(Q // QB, nk),
        in_specs=[
            pl.BlockSpec((QB, D), lambda qi, ki: (qi, 0)),
            pl.BlockSpec((KT, D), lambda qi, ki: (ki, 0)),
        ],
        out_specs=[
            pl.BlockSpec((QB * CPT, _CH),
                         lambda qi, ki, nk=nk: (qi * nk + ki, 0)),
            pl.BlockSpec((QB, _TOP_K), lambda qi, ki: (qi, 0)),
            pl.BlockSpec((QB, _TOP_K), lambda qi, ki: (qi, 0)),
            pl.BlockSpec((QB, _TOP_K), lambda qi, ki: (qi, 0)),
        ],
        out_shape=[
            jax.ShapeDtypeStruct((Q * (K // _CH), _CH), jnp.float32),
            jax.ShapeDtypeStruct((Q, _TOP_K), jnp.int32),
            jax.ShapeDtypeStruct((Q, _TOP_K), jnp.int32),
            jax.ShapeDtypeStruct((Q, _TOP_K), jnp.float32),
        ],
        scratch_shapes=[pltpu.VMEM((nk, QB, CPT), jnp.float32),
                        pltpu.VMEM((nk, KT), jnp.float32)],
        compiler_params=pltpu.CompilerParams(
            dimension_semantics=("parallel", "arbitrary")),
    )(h, db_keys)


def _run_retrieve(s_tbl, rows_flat, cs_flat, t_flat, db_values, Q, V, C):
    KDB = db_values.shape[0]
    NW = 32
    QW = Q // NW
    NCAND = _TOP_K * _CH
    mesh = plsc.VectorSubcoreMesh(core_axis_name="c", subcore_axis_name="s")

    @functools.partial(
        pl.kernel, mesh=mesh,
        out_type=jax.ShapeDtypeStruct((Q, V), jnp.float32),
        compiler_params=pltpu.CompilerParams(
            use_tc_tiling_on_sc=False, needs_layout_passes=False),
        scratch_types=[
            pltpu.VMEM((KDB,), jnp.int32),
            pltpu.VMEM((QW, _TOP_K), jnp.int32),
            pltpu.VMEM((QW * _TOP_K * 16,), jnp.int32),
            pltpu.VMEM((QW * 16,), jnp.float32),
            pltpu.VMEM((_TOP_K, _CH), jnp.float32),
            pltpu.VMEM((NCAND + 16,), jnp.float32),
            pltpu.VMEM((NCAND + 16,), jnp.int32),
            pltpu.VMEM((V,), jnp.float32),
            pltpu.SemaphoreType.DMA,
        ],
    )
    def sc_kernel(s_hbm, rows_hbm, cs_hbm, t_hbm, dbv_hbm, out_hbm,
                  dbv_v, rid_v, cs_v, t_v, cand_v, vbuf, ibuf, row_v, sem):
        wid = lax.axis_index("s") * 2 + lax.axis_index("c")
        base = wid * QW * _TOP_K
        pltpu.sync_copy(dbv_hbm, dbv_v)
        pltpu.sync_copy(rows_hbm.at[pl.ds(wid * QW, QW)], rid_v)
        pltpu.sync_copy(cs_hbm.at[pl.ds(base * 16, QW * _TOP_K * 16)], cs_v)
        pltpu.sync_copy(t_hbm.at[pl.ds(wid * QW * 16, QW * 16)], t_v)

        zeros16 = jnp.zeros((16,), jnp.float32)
        negs16 = jnp.full((16,), _NEG, jnp.float32)
        ones16 = jnp.ones((16,), jnp.bool_)
        lanes = lax.iota(jnp.int32, 16)

        def zbody(i, c):
            row_v[pl.ds(i * 16, 16)] = zeros16
            return c
        lax.fori_loop(0, V // 16, zbody, 0)

        def qbody(q, c):
            qg = wid * QW + q
            pltpu.async_copy(s_hbm.at[rid_v.at[q]], cand_v, sem).wait()
            tq = t_v[pl.ds(q * 16, 16)]
            cnt = 0
            for r in range(_TOP_K):
                cbase = cs_v[pl.ds((q * _TOP_K + r) * 16, 16)] * _CH
                for j in range(_CH // 16):
                    v = cand_v[r, pl.ds(j * 16, 16)]
                    gi = (cbase + j * 16) + lanes
                    mask = v >= tq
                    plsc.store_compressed(vbuf.at[pl.ds(cnt, 16)], v, mask=mask)
                    plsc.store_compressed(ibuf.at[pl.ds(cnt, 16)], gi, mask=mask)
                    cnt = cnt + jnp.sum(mask.astype(jnp.int32))
            plsc.store_compressed(vbuf.at[pl.ds(cnt, 16)], negs16, mask=ones16)
            nv = lax.shift_right_logical(cnt + 15, 4)
            msel0 = negs16
            msel1 = negs16
            isel0 = jnp.zeros((16,), jnp.int32)
            isel1 = jnp.zeros((16,), jnp.int32)
            for t in range(_TOP_K):
                def mbody(j, acc):
                    return jnp.maximum(acc, vbuf[pl.ds(j * 16, 16)])
                acc = lax.fori_loop(0, nv, mbody, negs16)
                mval = jnp.max(acc)

                def ibody(j, imin):
                    vj = vbuf[pl.ds(j * 16, 16)]
                    ij = ibuf[pl.ds(j * 16, 16)]
                    cand = jnp.where(vj == mval, ij, jnp.int32(2**30))
                    return jnp.minimum(imin, jnp.min(cand))
                imin = lax.fori_loop(0, nv, ibody, jnp.int32(2**30))

                def xbody(j, c2):
                    vj = vbuf[pl.ds(j * 16, 16)]
                    ij = ibuf[pl.ds(j * 16, 16)]
                    vbuf[pl.ds(j * 16, 16)] = jnp.where(
                        (vj == mval) & (ij == imin), _NEG, vj)
                    return c2
                lax.fori_loop(0, nv, xbody, 0)
                if t < 16:
                    msel0 = jnp.where(lanes == t, mval, msel0)
                    isel0 = jnp.where(lanes == t, imin, isel0)
                else:
                    msel1 = jnp.where(lanes == t - 16, mval, msel1)
                    isel1 = jnp.where(lanes == t - 16, imin, isel1)
            m0 = jnp.max(msel0)
            e0 = jnp.exp(msel0 - m0)
            e1 = jnp.exp(msel1 - m0)
            denom = jnp.zeros((16,), jnp.float32) + (jnp.sum(e0) + jnp.sum(e1))
            scale = _MIX / denom
            w0 = e0 * scale
            w1 = e1 * scale
            tok0 = plsc.load_gather(dbv_v, [isel0])
            tok1 = plsc.load_gather(dbv_v, [isel1])
            for l in range(16):
                plsc.addupdate_scatter(row_v, [tok0], w0, mask=lanes == l)
                plsc.addupdate_scatter(row_v, [tok1], w1, mask=lanes == l)
            rb0 = plsc.load_gather(row_v, [tok0])
            dep = jnp.min(rb0 * 0.0).astype(jnp.int32)
            pltpu.sync_copy(row_v, out_hbm.at[qg + dep])
            plsc.store_scatter(row_v, [tok0], zeros16)
            plsc.store_scatter(row_v, [tok1], zeros16)
            return c
        lax.fori_loop(0, QW, qbody, 0)

    return sc_kernel(s_tbl, rows_flat, cs_flat, t_flat, db_values)


def _mix_body(lg_ref, ebd_ref, o_ref):
    lg = lg_ref[...]
    m = jnp.max(lg, axis=1, keepdims=True)
    e = jnp.exp(lg - m)
    z = jnp.sum(e, axis=1, keepdims=True)
    o_ref[...] = jnp.log(((1.0 - _MIX) / z) * e + ebd_ref[...])


def _run_mix(lg, ebd):
    Q, V = lg.shape
    QB = 32
    return pl.pallas_call(
        _mix_body,
        grid=(Q // QB,),
        in_specs=[
            pl.BlockSpec((QB, V), lambda qi: (qi, 0)),
            pl.BlockSpec((QB, V), lambda qi: (qi, 0)),
        ],
        out_specs=pl.BlockSpec((QB, V), lambda qi: (qi, 0)),
        out_shape=jax.ShapeDtypeStruct((Q, V), jnp.float32),
        compiler_params=pltpu.CompilerParams(
            dimension_semantics=("parallel",)),
    )(lg, ebd)


def _half_pipeline(h_half, db_keys, db_values, V):
    Qh, _ = h_half.shape
    K = db_keys.shape[0]
    C = K // _CH
    s, rows, csel, tvals = _run_scores(h_half, db_keys)
    cs16 = jnp.broadcast_to(
        csel.reshape(-1, 1), (Qh * _TOP_K, 16)).reshape(-1)
    t16 = jnp.broadcast_to(
        tvals[:, _TOP_K - 1:_TOP_K], (Qh, 16)).reshape(-1)
    return _run_retrieve(s, rows, cs16, t16, db_values, Qh, V, C)


def kernel(hidden, logits, db_keys, db_values):
    B, T, D = hidden.shape
    V = logits.shape[-1]
    Q = B * T
    h = hidden.reshape(Q, D)
    lg = logits.reshape(Q, V)
    nsplit = 4
    Qs = Q // nsplit
    ebd = jnp.zeros((Q, V), jnp.float32)
    for i in range(nsplit):
        e = _half_pipeline(h[i * Qs:(i + 1) * Qs], db_keys, db_values, V)
        ebd = lax.dynamic_update_slice(ebd, e, (i * Qs, 0))
    out = _run_mix(lg, ebd)
    return out.reshape(B, T, V)
